# baseline (speedup 1.0000x reference)
import functools

import jax
import jax.numpy as jnp
from jax.experimental import pallas as pl
from jax.experimental.pallas import tpu as pltpu


def _top2_combine(logits, e):
    iota = jax.lax.broadcasted_iota(jnp.int32, logits.shape, 1)
    m1 = jnp.max(logits, axis=-1, keepdims=True)
    i1 = jnp.min(jnp.where(logits == m1, iota, e), axis=-1, keepdims=True)
    masked = jnp.where(iota == i1, -jnp.inf, logits)
    m2 = jnp.max(masked, axis=-1, keepdims=True)
    i2 = jnp.min(jnp.where(masked == m2, iota, e), axis=-1, keepdims=True)
    p1 = jax.nn.sigmoid(m1 - m2)
    p2 = 1.0 - p1
    return jnp.where(iota == i1, p1, 0.0) + jnp.where(iota == i2, p2, 0.0)


def _moe_kernel(x_ref, rw_ref, brw_ref, wbank_ref, bbank_ref, out_ref, wc_ref,
                *, n_experts):
    e = pl.program_id(0)

    @pl.when(e == 0)
    def _init():
        x = x_ref[...]
        w_logits = jnp.dot(x, rw_ref[...], preferred_element_type=jnp.float32)
        b_logits = jnp.dot(x, brw_ref[...], preferred_element_type=jnp.float32)
        wc_ref[...] = _top2_combine(w_logits, n_experts)
        b_comb = _top2_combine(b_logits, n_experts)
        out_ref[...] = jnp.dot(b_comb, bbank_ref[...],
                               preferred_element_type=jnp.float32)

    y = jnp.dot(x_ref[...], wbank_ref[0], preferred_element_type=jnp.float32)
    wc = wc_ref[...]
    iota = jax.lax.broadcasted_iota(jnp.int32, wc.shape, 1)
    wcol = jnp.sum(jnp.where(iota == e, wc, 0.0), axis=1, keepdims=True)
    out_ref[...] += wcol * y


@jax.jit
def kernel(input_batch, router_w, bias_router_w, weight_bank, bias_bank):
    b, d = input_batch.shape
    e, _, o = weight_bank.shape
    return pl.pallas_call(
        functools.partial(_moe_kernel, n_experts=e),
        grid=(e,),
        out_shape=jax.ShapeDtypeStruct((b, o), jnp.float32),
        in_specs=[
            pl.BlockSpec((b, d), lambda i: (0, 0)),
            pl.BlockSpec((d, e), lambda i: (0, 0)),
            pl.BlockSpec((d, e), lambda i: (0, 0)),
            pl.BlockSpec((1, d, o), lambda i: (i, 0, 0)),
            pl.BlockSpec((e, o), lambda i: (0, 0)),
        ],
        out_specs=pl.BlockSpec((b, o), lambda i: (0, 0)),
        scratch_shapes=[pltpu.VMEM((b, e), jnp.float32)],
    )(input_batch, router_w, bias_router_w, weight_bank, bias_bank)


# speedup vs baseline: 1.3796x; 1.3796x over previous
import functools

import jax
import jax.numpy as jnp
from jax.experimental import pallas as pl
from jax.experimental.pallas import tpu as pltpu


def _top2_combine(logits, e):
    iota = jax.lax.broadcasted_iota(jnp.int32, logits.shape, 1)
    m1 = jnp.max(logits, axis=-1, keepdims=True)
    i1 = jnp.min(jnp.where(logits == m1, iota, e), axis=-1, keepdims=True)
    masked = jnp.where(iota == i1, -jnp.inf, logits)
    m2 = jnp.max(masked, axis=-1, keepdims=True)
    i2 = jnp.min(jnp.where(masked == m2, iota, e), axis=-1, keepdims=True)
    p1 = jax.nn.sigmoid(m1 - m2)
    p2 = 1.0 - p1
    return jnp.where(iota == i1, p1, 0.0) + jnp.where(iota == i2, p2, 0.0)


def _moe_kernel(x_ref, rw_ref, brw_ref, wbank_hbm, bbank_ref, out_ref,
                wbuf, sems, *, n_experts):
    for e in range(n_experts):
        pltpu.make_async_copy(wbank_hbm.at[e], wbuf.at[e], sems.at[e]).start()

    x = x_ref[...]
    w_logits = jnp.dot(x, rw_ref[...], preferred_element_type=jnp.float32)
    b_logits = jnp.dot(x, brw_ref[...], preferred_element_type=jnp.float32)
    w_comb = _top2_combine(w_logits, n_experts)
    b_comb = _top2_combine(b_logits, n_experts)

    acc = jnp.dot(b_comb, bbank_ref[...], preferred_element_type=jnp.float32)
    for e in range(n_experts):
        pltpu.make_async_copy(wbank_hbm.at[e], wbuf.at[e], sems.at[e]).wait()
        y = jnp.dot(x, wbuf[e], preferred_element_type=jnp.float32)
        acc = acc + w_comb[:, e][:, None] * y
    out_ref[...] = acc


@jax.jit
def kernel(input_batch, router_w, bias_router_w, weight_bank, bias_bank):
    b, d = input_batch.shape
    e, _, o = weight_bank.shape
    return pl.pallas_call(
        functools.partial(_moe_kernel, n_experts=e),
        out_shape=jax.ShapeDtypeStruct((b, o), jnp.float32),
        in_specs=[
            pl.BlockSpec((b, d), lambda: (0, 0)),
            pl.BlockSpec((d, e), lambda: (0, 0)),
            pl.BlockSpec((d, e), lambda: (0, 0)),
            pl.BlockSpec(memory_space=pl.ANY),
            pl.BlockSpec((e, o), lambda: (0, 0)),
        ],
        out_specs=pl.BlockSpec((b, o), lambda: (0, 0)),
        scratch_shapes=[
            pltpu.VMEM((e, d, o), jnp.float32),
            pltpu.SemaphoreType.DMA((e,)),
        ],
    )(input_batch, router_w, bias_router_w, weight_bank, bias_bank)
